# Initial kernel scaffold; baseline (speedup 1.0000x reference)
import jax
import jax.numpy as jnp
from jax.experimental import pallas as pl


def kernel(tok_idx, token_table, pos_table):
    raise NotImplementedError("write your pallas kernel here")


# speedup vs baseline: 1.1614x; 1.1614x over previous
import functools

import jax
import jax.numpy as jnp
from jax import lax
from jax.experimental import pallas as pl
from jax.experimental.pallas import tpu as pltpu
from jax.experimental.pallas import tpu_sc as plsc

VOCAB = 100000
EMBED = 768
CTX = 2048
B = 4
S = 2048

NUM_CORES = 2
NUM_SUBCORES = 16
NUM_WORKERS = NUM_CORES * NUM_SUBCORES
S_BLK = S // NUM_WORKERS
LANES = 16
COL_CHUNKS = EMBED // LANES


def _emb_kernel(idx_hbm, tok_hbm, pos_hbm, out_hbm, idx_v, pos_v, rows_v, sem):
    wid = lax.axis_index("s") * NUM_CORES + lax.axis_index("c")
    s0 = wid * S_BLK

    pltpu.sync_copy(pos_hbm.at[pl.ds(s0, S_BLK)], pos_v)

    for b in range(B):
        base = b * S + s0
        pltpu.sync_copy(idx_hbm.at[pl.ds(base, S_BLK)], idx_v)
        pltpu.async_copy(tok_hbm.at[idx_v], rows_v, sem).wait()

        def row_body(r, carry):
            for j in range(COL_CHUNKS):
                sl = pl.ds(j * LANES, LANES)
                rows_v[r, sl] = rows_v[r, sl] + pos_v[r, sl]
            return carry

        lax.fori_loop(0, S_BLK, row_body, 0)
        pltpu.sync_copy(rows_v, out_hbm.at[pl.ds(base, S_BLK)])


@jax.jit
def _run(idx_flat, token_table, pos_table):
    mesh = plsc.VectorSubcoreMesh(core_axis_name="c", subcore_axis_name="s")
    f = functools.partial(
        pl.kernel,
        mesh=mesh,
        out_type=jax.ShapeDtypeStruct((B * S, EMBED), jnp.float32),
        scratch_types=[
            pltpu.VMEM((S_BLK,), jnp.int32),
            pltpu.VMEM((S_BLK, EMBED), jnp.float32),
            pltpu.VMEM((S_BLK, EMBED), jnp.float32),
            pltpu.SemaphoreType.DMA,
        ],
    )(_emb_kernel)
    return f(idx_flat, token_table, pos_table)


def kernel(tok_idx, token_table, pos_table):
    idx_flat = tok_idx.reshape(-1).astype(jnp.int32)
    out = _run(idx_flat, token_table, pos_table)
    return out.reshape(B, S, EMBED)
